# baseline (speedup 1.0000x reference)
import functools

import jax
import jax.numpy as jnp
from jax import lax
from jax.experimental import pallas as pl
from jax.experimental.pallas import tpu as pltpu
from jax.experimental.pallas import tpu_sc as plsc

BIN = 100
PAD = 128
HID = 128
TB = 8192


def _idx_body(x_ref, w1_ref, b1_ref, w2t_ref, b2_ref, pw_ref, idx_ref):
    xr = x_ref[0]
    h1 = jnp.maximum(w1_ref[...] * xr + b1_ref[...], 0.0)
    h2 = jax.lax.dot_general(
        w2t_ref[...], h1, (((1,), (0,)), ((), ())),
        precision=jax.lax.Precision.DEFAULT,
        preferred_element_type=jnp.float32,
    ) + b2_ref[...]
    h2 = jnp.maximum(h2, 0.0)
    m = jnp.max(h2, axis=0, keepdims=True)
    sel = (h2 >= m).astype(jnp.float32)
    rowval = jax.lax.dot_general(
        pw_ref[...], sel, (((1,), (0,)), ((), ())),
        precision=jax.lax.Precision.DEFAULT,
        preferred_element_type=jnp.float32,
    )
    bits = jax.lax.bitcast_convert_type(rowval, jnp.int32)
    idx = jnp.maximum(127 - (bits >> 23), 0)
    idx_ref[0] = idx


def _compute_idx(x2, W1, b1, W2, b2):
    G = x2.shape[0]
    w1c = jnp.zeros((PAD, 1), jnp.float32).at[:BIN, 0].set(W1[0])
    b1c = jnp.zeros((PAD, 1), jnp.float32).at[:BIN, 0].set(b1)
    w2t = jnp.zeros((PAD, PAD), jnp.float32).at[:BIN, :BIN].set(W2.T)
    b2c = jnp.zeros((PAD, 1), jnp.float32).at[:BIN, 0].set(b2)
    binr = jnp.arange(PAD, dtype=jnp.float32)
    pw = jnp.where(binr < BIN, jnp.exp2(-binr), 0.0).reshape(1, PAD)
    idx = pl.pallas_call(
        _idx_body,
        grid=(G,),
        in_specs=[
            pl.BlockSpec((1, 1, TB), lambda i: (i, 0, 0)),
            pl.BlockSpec((PAD, 1), lambda i: (0, 0)),
            pl.BlockSpec((PAD, 1), lambda i: (0, 0)),
            pl.BlockSpec((PAD, PAD), lambda i: (0, 0)),
            pl.BlockSpec((PAD, 1), lambda i: (0, 0)),
            pl.BlockSpec((1, PAD), lambda i: (0, 0)),
        ],
        out_specs=pl.BlockSpec((1, 1, TB), lambda i: (i, 0, 0)),
        out_shape=jax.ShapeDtypeStruct((G, 1, TB), jnp.int32),
    )(x2.reshape(G, 1, TB), w1c, b1c, w2t, b2c, pw)
    return idx.reshape(G * TB)


_INFO = plsc.get_sparse_core_info()
_NC, _NS = _INFO.num_cores, _INFO.num_subcores
_NW = _NC * _NS
_CHUNK = 128


_NSLOT = 4


def _make_sc_gather(N):
    b_per_w = N // _NW
    n_rounds = b_per_w // (_NSLOT * _CHUNK)
    mesh = plsc.VectorSubcoreMesh(core_axis_name="c", subcore_axis_name="s")

    @functools.partial(
        pl.kernel, mesh=mesh,
        out_type=jax.ShapeDtypeStruct((N, HID), jnp.float32),
        scratch_types=(
            [pltpu.VMEM_SHARED((BIN, HID), jnp.float32)]
            + [pltpu.VMEM((_CHUNK,), jnp.int32)] * _NSLOT
            + [pltpu.VMEM((_CHUNK, HID), jnp.float32)] * _NSLOT
            + [pltpu.SemaphoreType.DMA] * (2 * _NSLOT)
        ),
        compiler_params=pltpu.CompilerParams(needs_layout_passes=False),
    )
    def sc_gather(emb_hbm, idx_hbm, out_hbm, emb_sh, *bufs):
        idx_v = bufs[:_NSLOT]
        rows_v = bufs[_NSLOT:2 * _NSLOT]
        gsem = bufs[2 * _NSLOT:3 * _NSLOT]
        ssem = bufs[3 * _NSLOT:4 * _NSLOT]
        sid = lax.axis_index("s")
        wid = sid * _NC + lax.axis_index("c")
        base = wid * b_per_w

        @pl.when(sid == 0)
        def _():
            pltpu.sync_copy(emb_hbm, emb_sh)

        plsc.subcore_barrier()

        def issue(s, off):
            pltpu.sync_copy(idx_hbm.at[pl.ds(off, _CHUNK)], idx_v[s])
            pltpu.async_copy(emb_sh.at[idx_v[s]], rows_v[s], gsem[s])

        def flush(s, off):
            pltpu.make_async_copy(emb_sh.at[idx_v[s]], rows_v[s],
                                  gsem[s]).wait()
            pltpu.async_copy(rows_v[s], out_hbm.at[pl.ds(off, _CHUNK)],
                             ssem[s])

        def sdrain(s):
            pltpu.make_async_copy(
                rows_v[s], out_hbm.at[pl.ds(0, _CHUNK)], ssem[s]).wait()

        for s in range(_NSLOT):
            issue(s, base + s * _CHUNK)

        def round_(p, carry):
            prev = base + (p - 1) * (_NSLOT * _CHUNK)
            cur = base + p * (_NSLOT * _CHUNK)
            for s in range(_NSLOT):
                flush(s, prev + s * _CHUNK)
            for s in range(_NSLOT):
                sdrain(s)
                issue(s, cur + s * _CHUNK)
            return carry

        lax.fori_loop(1, n_rounds, round_, 0)
        last = base + (n_rounds - 1) * (_NSLOT * _CHUNK)
        for s in range(_NSLOT):
            flush(s, last + s * _CHUNK)
        for s in range(_NSLOT):
            sdrain(s)

    return sc_gather


def kernel(x, W1, b1, W2, b2, emb):
    B, L, _ = x.shape
    N = B * L
    x2 = x.reshape(N // TB, TB)
    idx = _compute_idx(x2, W1, b1, W2, b2)
    out = _make_sc_gather(N)(emb, idx)
    return out.reshape(B, L, HID)


# speedup vs baseline: 1.0072x; 1.0072x over previous
import functools

import jax
import jax.numpy as jnp
from jax import lax
from jax.experimental import pallas as pl
from jax.experimental.pallas import tpu as pltpu
from jax.experimental.pallas import tpu_sc as plsc

BIN = 100
PAD = 128
HID = 128
TB = 4096


def _idx_body(x_ref, w1_ref, b1_ref, w2t_ref, b2_ref, pw_ref, idx_ref):
    xr = x_ref[0]
    h1 = jnp.maximum(w1_ref[...] * xr + b1_ref[...], 0.0)
    h2 = jax.lax.dot_general(
        w2t_ref[...], h1, (((1,), (0,)), ((), ())),
        precision=jax.lax.Precision.DEFAULT,
        preferred_element_type=jnp.float32,
    ) + b2_ref[...]
    h2 = jnp.maximum(h2, 0.0)
    m = jnp.max(h2, axis=0, keepdims=True)
    sel = (h2 >= m).astype(jnp.float32)
    rowval = jax.lax.dot_general(
        pw_ref[...], sel, (((1,), (0,)), ((), ())),
        precision=jax.lax.Precision.DEFAULT,
        preferred_element_type=jnp.float32,
    )
    bits = jax.lax.bitcast_convert_type(rowval, jnp.int32)
    idx = jnp.maximum(127 - (bits >> 23), 0)
    idx_ref[0] = idx


def _compute_idx(x2, W1, b1, W2, b2):
    G = x2.shape[0]
    w1c = jnp.zeros((PAD, 1), jnp.float32).at[:BIN, 0].set(W1[0])
    b1c = jnp.zeros((PAD, 1), jnp.float32).at[:BIN, 0].set(b1)
    w2t = jnp.zeros((PAD, PAD), jnp.float32).at[:BIN, :BIN].set(W2.T)
    b2c = jnp.zeros((PAD, 1), jnp.float32).at[:BIN, 0].set(b2)
    binr = jnp.arange(PAD, dtype=jnp.float32)
    pw = jnp.where(binr < BIN, jnp.exp2(-binr), 0.0).reshape(1, PAD)
    idx = pl.pallas_call(
        _idx_body,
        grid=(G,),
        in_specs=[
            pl.BlockSpec((1, 1, TB), lambda i: (i, 0, 0)),
            pl.BlockSpec((PAD, 1), lambda i: (0, 0)),
            pl.BlockSpec((PAD, 1), lambda i: (0, 0)),
            pl.BlockSpec((PAD, PAD), lambda i: (0, 0)),
            pl.BlockSpec((PAD, 1), lambda i: (0, 0)),
            pl.BlockSpec((1, PAD), lambda i: (0, 0)),
        ],
        out_specs=pl.BlockSpec((1, 1, TB), lambda i: (i, 0, 0)),
        out_shape=jax.ShapeDtypeStruct((G, 1, TB), jnp.int32),
    )(x2.reshape(G, 1, TB), w1c, b1c, w2t, b2c, pw)
    return idx.reshape(G * TB)


_INFO = plsc.get_sparse_core_info()
_NC, _NS = _INFO.num_cores, _INFO.num_subcores
_NW = _NC * _NS
_CHUNK = 128


_NSLOT = 4


def _make_sc_gather(N):
    b_per_w = N // _NW
    n_rounds = b_per_w // (_NSLOT * _CHUNK)
    mesh = plsc.VectorSubcoreMesh(core_axis_name="c", subcore_axis_name="s")

    @functools.partial(
        pl.kernel, mesh=mesh,
        out_type=jax.ShapeDtypeStruct((N, HID), jnp.float32),
        scratch_types=(
            [pltpu.VMEM_SHARED((BIN, HID), jnp.float32)]
            + [pltpu.VMEM((_CHUNK,), jnp.int32)] * _NSLOT
            + [pltpu.VMEM((_CHUNK, HID), jnp.float32)] * _NSLOT
            + [pltpu.SemaphoreType.DMA] * (2 * _NSLOT)
        ),
        compiler_params=pltpu.CompilerParams(needs_layout_passes=False),
    )
    def sc_gather(emb_hbm, idx_hbm, out_hbm, emb_sh, *bufs):
        idx_v = bufs[:_NSLOT]
        rows_v = bufs[_NSLOT:2 * _NSLOT]
        gsem = bufs[2 * _NSLOT:3 * _NSLOT]
        ssem = bufs[3 * _NSLOT:4 * _NSLOT]
        sid = lax.axis_index("s")
        wid = sid * _NC + lax.axis_index("c")
        base = wid * b_per_w

        @pl.when(sid == 0)
        def _():
            pltpu.sync_copy(emb_hbm, emb_sh)

        plsc.subcore_barrier()

        def issue(s, off):
            pltpu.sync_copy(idx_hbm.at[pl.ds(off, _CHUNK)], idx_v[s])
            pltpu.async_copy(emb_sh.at[idx_v[s]], rows_v[s], gsem[s])

        def flush(s, off):
            pltpu.make_async_copy(emb_sh.at[idx_v[s]], rows_v[s],
                                  gsem[s]).wait()
            pltpu.async_copy(rows_v[s], out_hbm.at[pl.ds(off, _CHUNK)],
                             ssem[s])

        def sdrain(s):
            pltpu.make_async_copy(
                rows_v[s], out_hbm.at[pl.ds(0, _CHUNK)], ssem[s]).wait()

        for s in range(_NSLOT):
            issue(s, base + s * _CHUNK)

        def round_(p, carry):
            prev = base + (p - 1) * (_NSLOT * _CHUNK)
            cur = base + p * (_NSLOT * _CHUNK)
            for s in range(_NSLOT):
                flush(s, prev + s * _CHUNK)
            for s in range(_NSLOT):
                sdrain(s)
                issue(s, cur + s * _CHUNK)
            return carry

        lax.fori_loop(1, n_rounds, round_, 0)
        last = base + (n_rounds - 1) * (_NSLOT * _CHUNK)
        for s in range(_NSLOT):
            flush(s, last + s * _CHUNK)
        for s in range(_NSLOT):
            sdrain(s)

    return sc_gather


def kernel(x, W1, b1, W2, b2, emb):
    B, L, _ = x.shape
    N = B * L
    x2 = x.reshape(N // TB, TB)
    idx = _compute_idx(x2, W1, b1, W2, b2)
    out = _make_sc_gather(N)(emb, idx)
    return out.reshape(B, L, HID)
